# baseline (speedup 1.0000x reference)
import jax
import jax.numpy as jnp
from jax import lax
from jax.experimental import pallas as pl
from jax.experimental.pallas import tpu as pltpu
from jax.experimental.pallas import tpu_sc as plsc

VOCAB = 100000
DIM = 128
B = 4096
CTX = 20
NOISE = 10

NUM_CORES = 2
NUM_SUBCORES = 16
NW = NUM_CORES * NUM_SUBCORES
BPW = B // NW
JPW = BPW * NOISE
BCH = 16
CHUNK = BCH * NOISE
NCHUNK = JPW // CHUNK

_SC_PARAMS = pltpu.CompilerParams(needs_layout_passes=False)


def _sc_body(ctxT_hbm, doc_hbm, tnT_hbm, wemb_hbm, lect_hbm, ot_hbm, outT_hbm,
             ctx_v, doc_v, tnt_v, tn_v, x_v, x2_v, rows_a, rows_b, pad_v,
             outT_v, sem1, sem2, sem_a, sem_b):
    cid = lax.axis_index("c")
    sid = lax.axis_index("s")
    base = (sid * NUM_CORES + cid) * BPW

    iota16 = lax.iota(jnp.int32, 16)

    pltpu.sync_copy(ctxT_hbm.at[:, pl.ds(base, BPW)], ctx_v)
    pltpu.sync_copy(doc_hbm.at[pl.ds(base, BPW)], doc_v)
    pltpu.sync_copy(tnT_hbm.at[:, pl.ds(base, BPW)], tnt_v)

    d0 = pltpu.async_copy(lect_hbm.at[doc_v], x_v, sem1)
    d1 = pltpu.async_copy(wemb_hbm.at[ctx_v.at[0]], x2_v, sem1)
    d0.wait()
    d1.wait()
    descs = []
    for c in range(1, CTX):
        dst = x_v if c % 2 else x2_v
        descs.append(
            pltpu.async_copy(wemb_hbm.at[ctx_v.at[c]], dst, sem2, add=True))

    def tbody(g, _):
        j16 = g * 16 + iota16
        b16 = j16 // NOISE
        n16 = j16 - b16 * NOISE
        tn_v[pl.ds(g * 16, 16)] = plsc.load_gather(tnt_v, [n16, b16])
        return 0

    lax.fori_loop(0, JPW // 16, tbody, 0)

    pltpu.async_copy(ot_hbm.at[tn_v.at[pl.ds(0, CHUNK)]], rows_a, sem_a)
    pltpu.async_copy(ot_hbm.at[tn_v.at[pl.ds(CHUNK, CHUNK)]], rows_b, sem_b)

    for d in descs:
        d.wait()
    zero16 = jnp.zeros((16,), jnp.float32)
    masks = [iota16 == n for n in range(NOISE)]
    gidx = [iota16 * 16 + n for n in range(NOISE)]

    def compute(ch, rows_v):
        def bbody(bb, _):
            b = ch * BCH + bb
            xr = [x_v[b, pl.ds(k * 16, 16)] + x2_v[b, pl.ds(k * 16, 16)]
                  for k in range(DIM // 16)]
            out16 = zero16
            for n in range(NOISE):
                jj = bb * NOISE + n
                acc = xr[0] * rows_v[jj, pl.ds(0, 16)]
                for k in range(1, DIM // 16):
                    acc = acc + xr[k] * rows_v[jj, pl.ds(k * 16, 16)]
                out16 = jnp.where(masks[n], jnp.sum(acc), out16)
            pad_v[pl.ds(bb * 16, 16)] = out16
            return 0

        lax.fori_loop(0, BCH, bbody, 0)
        for n in range(NOISE):
            outT_v[n, pl.ds(ch * BCH, BCH)] = plsc.load_gather(pad_v, [gidx[n]])

    def wait_rows(rows_v, sem):
        pltpu.make_async_copy(ot_hbm.at[pl.ds(0, CHUNK), :], rows_v, sem).wait()

    def pair_body(p, _):
        ch_a = 2 * p
        wait_rows(rows_a, sem_a)
        compute(ch_a, rows_a)

        @pl.when(p < NCHUNK // 2 - 1)
        def _():
            pltpu.async_copy(
                ot_hbm.at[tn_v.at[pl.ds((ch_a + 2) * CHUNK, CHUNK)]],
                rows_a, sem_a)

        wait_rows(rows_b, sem_b)
        compute(ch_a + 1, rows_b)

        @pl.when(p < NCHUNK // 2 - 1)
        def _():
            pltpu.async_copy(
                ot_hbm.at[tn_v.at[pl.ds((ch_a + 3) * CHUNK, CHUNK)]],
                rows_b, sem_b)

        return 0

    lax.fori_loop(0, NCHUNK // 2, pair_body, 0)
    pltpu.sync_copy(outT_v, outT_hbm.at[:, pl.ds(base, BPW)])


@jax.jit
def _sc_call(ctxT, doc_ids, tnT, word_emb, lecture, ot):
    mesh = plsc.VectorSubcoreMesh(core_axis_name="c", subcore_axis_name="s")
    f = pl.kernel(
        _sc_body,
        out_type=jax.ShapeDtypeStruct((NOISE, B), jnp.float32),
        mesh=mesh,
        compiler_params=_SC_PARAMS,
        scratch_types=[
            pltpu.VMEM((CTX, BPW), jnp.int32),
            pltpu.VMEM((BPW,), jnp.int32),
            pltpu.VMEM((NOISE, BPW), jnp.int32),
            pltpu.VMEM((JPW,), jnp.int32),
            pltpu.VMEM((BPW, DIM), jnp.float32),
            pltpu.VMEM((BPW, DIM), jnp.float32),
            pltpu.VMEM((CHUNK, DIM), jnp.float32),
            pltpu.VMEM((CHUNK, DIM), jnp.float32),
            pltpu.VMEM((BCH * 16,), jnp.float32),
            pltpu.VMEM((NOISE, BPW), jnp.float32),
            pltpu.SemaphoreType.DMA,
            pltpu.SemaphoreType.DMA,
            pltpu.SemaphoreType.DMA,
            pltpu.SemaphoreType.DMA,
        ],
    )
    return f(ctxT, doc_ids, tnT, word_emb, lecture, ot)


def kernel(context_ids, doc_ids, target_noise_ids, word_emb, lecture, O):
    outT = _sc_call(context_ids.T, doc_ids, target_noise_ids.T,
                    word_emb, lecture, O.T)
    return outT.T


# speedup vs baseline: 1.0239x; 1.0239x over previous
import jax
import jax.numpy as jnp
from jax import lax
from jax.experimental import pallas as pl
from jax.experimental.pallas import tpu as pltpu
from jax.experimental.pallas import tpu_sc as plsc

VOCAB = 100000
DIM = 128
B = 4096
CTX = 20
NOISE = 10

NUM_CORES = 2
NUM_SUBCORES = 16
NW = NUM_CORES * NUM_SUBCORES
BPW = B // NW
JPW = BPW * NOISE
BCH = 16
CHUNK = BCH * NOISE
NCHUNK = JPW // CHUNK

_SC_PARAMS = pltpu.CompilerParams(needs_layout_passes=False)


def _sc_body(ctxT_hbm, doc_hbm, tnT_hbm, wemb_hbm, lect_hbm, ot_hbm, outT_hbm,
             ctx_v, doc_v, tnt_v, tn_v, x_v, rows_a, rows_b, pad_v,
             outT_v, sem1, sem2, sem_a, sem_b):
    cid = lax.axis_index("c")
    sid = lax.axis_index("s")
    base = (sid * NUM_CORES + cid) * BPW

    iota16 = lax.iota(jnp.int32, 16)

    pltpu.sync_copy(ctxT_hbm.at[:, pl.ds(base, BPW)], ctx_v)
    pltpu.sync_copy(doc_hbm.at[pl.ds(base, BPW)], doc_v)
    pltpu.sync_copy(tnT_hbm.at[:, pl.ds(base, BPW)], tnt_v)

    zero16 = jnp.zeros((16,), jnp.float32)

    def zbody(i, _):
        for k in range(DIM // 16):
            x_v[i, pl.ds(k * 16, 16)] = zero16
        return 0

    lax.fori_loop(0, BPW, zbody, 0)
    descs = [pltpu.async_copy(lect_hbm.at[doc_v], x_v, sem2, add=True)]
    for c in range(CTX):
        descs.append(
            pltpu.async_copy(wemb_hbm.at[ctx_v.at[c]], x_v, sem2, add=True))

    def tbody(g, _):
        j16 = g * 16 + iota16
        b16 = j16 // NOISE
        n16 = j16 - b16 * NOISE
        tn_v[pl.ds(g * 16, 16)] = plsc.load_gather(tnt_v, [n16, b16])
        return 0

    lax.fori_loop(0, JPW // 16, tbody, 0)

    pltpu.async_copy(ot_hbm.at[tn_v.at[pl.ds(0, CHUNK)]], rows_a, sem_a)
    pltpu.async_copy(ot_hbm.at[tn_v.at[pl.ds(CHUNK, CHUNK)]], rows_b, sem_b)

    for d in descs:
        d.wait()
    masks = [iota16 == n for n in range(NOISE)]
    gidx = [iota16 * 16 + n for n in range(NOISE)]

    def compute(ch, rows_v):
        def bbody(bb, _):
            b = ch * BCH + bb
            xr = [x_v[b, pl.ds(k * 16, 16)] for k in range(DIM // 16)]
            out16 = zero16
            for n in range(NOISE):
                jj = bb * NOISE + n
                acc = xr[0] * rows_v[jj, pl.ds(0, 16)]
                for k in range(1, DIM // 16):
                    acc = acc + xr[k] * rows_v[jj, pl.ds(k * 16, 16)]
                out16 = jnp.where(masks[n], jnp.sum(acc), out16)
            pad_v[pl.ds(bb * 16, 16)] = out16
            return 0

        lax.fori_loop(0, BCH, bbody, 0)
        for n in range(NOISE):
            outT_v[n, pl.ds(ch * BCH, BCH)] = plsc.load_gather(pad_v, [gidx[n]])

    def wait_rows(rows_v, sem):
        pltpu.make_async_copy(ot_hbm.at[pl.ds(0, CHUNK), :], rows_v, sem).wait()

    def pair_body(p, _):
        ch_a = 2 * p
        wait_rows(rows_a, sem_a)
        compute(ch_a, rows_a)

        @pl.when(p < NCHUNK // 2 - 1)
        def _():
            pltpu.async_copy(
                ot_hbm.at[tn_v.at[pl.ds((ch_a + 2) * CHUNK, CHUNK)]],
                rows_a, sem_a)

        wait_rows(rows_b, sem_b)
        compute(ch_a + 1, rows_b)

        @pl.when(p < NCHUNK // 2 - 1)
        def _():
            pltpu.async_copy(
                ot_hbm.at[tn_v.at[pl.ds((ch_a + 3) * CHUNK, CHUNK)]],
                rows_b, sem_b)

        return 0

    lax.fori_loop(0, NCHUNK // 2, pair_body, 0)
    pltpu.sync_copy(outT_v, outT_hbm.at[:, pl.ds(base, BPW)])


@jax.jit
def _sc_call(ctxT, doc_ids, tnT, word_emb, lecture, ot):
    mesh = plsc.VectorSubcoreMesh(core_axis_name="c", subcore_axis_name="s")
    f = pl.kernel(
        _sc_body,
        out_type=jax.ShapeDtypeStruct((NOISE, B), jnp.float32),
        mesh=mesh,
        compiler_params=_SC_PARAMS,
        scratch_types=[
            pltpu.VMEM((CTX, BPW), jnp.int32),
            pltpu.VMEM((BPW,), jnp.int32),
            pltpu.VMEM((NOISE, BPW), jnp.int32),
            pltpu.VMEM((JPW,), jnp.int32),
            pltpu.VMEM((BPW, DIM), jnp.float32),
            pltpu.VMEM((CHUNK, DIM), jnp.float32),
            pltpu.VMEM((CHUNK, DIM), jnp.float32),
            pltpu.VMEM((BCH * 16,), jnp.float32),
            pltpu.VMEM((NOISE, BPW), jnp.float32),
            pltpu.SemaphoreType.DMA,
            pltpu.SemaphoreType.DMA,
            pltpu.SemaphoreType.DMA,
            pltpu.SemaphoreType.DMA,
        ],
    )
    return f(ctxT, doc_ids, tnT, word_emb, lecture, ot)


def kernel(context_ids, doc_ids, target_noise_ids, word_emb, lecture, O):
    outT = _sc_call(context_ids.T, doc_ids, target_noise_ids.T,
                    word_emb, lecture, O.T)
    return outT.T
